# baseline (speedup 1.0000x reference)
import functools

import jax
import jax.numpy as jnp
from jax import lax
from jax.experimental import pallas as pl
from jax.experimental.pallas import tpu as pltpu
from jax.experimental.pallas import tpu_sc as plsc

VOCAB = 1000000
D = 64
B = 4096
H = 50
LANES = 16
D_VREGS = D // LANES

NC = 2
NS = 16
NW = NC * NS
ROWS_PER_W = B // NW
RV = ROWS_PER_W // LANES
NACC = 8
K = 4
CHS = (290048, 236672, 236672, 236608)
CBASE = (0, 290048, 526720, 763392)


def _sc_pool_sums(xt, chunks):
  mesh = plsc.VectorSubcoreMesh(
      core_axis_name="c", subcore_axis_name="s", num_cores=NC, num_subcores=NS
  )

  @functools.partial(
      pl.kernel,
      out_type=jax.ShapeDtypeStruct((B, D), jnp.float32),
      mesh=mesh,
      compiler_params=pltpu.CompilerParams(use_tc_tiling_on_sc=False),
      scratch_types=[
          pltpu.VMEM((H, ROWS_PER_W), jnp.int32),
          pltpu.VMEM((K, H, ROWS_PER_W), jnp.int32),
          pltpu.VMEM((NACC, ROWS_PER_W, D), jnp.float32),
          [pltpu.SemaphoreType.DMA] * NACC,
      ],
  )
  def k(x_hbm, *refs):
    tabs = refs[:K]
    out_hbm = refs[K]
    idx_v, idxk_v, acc_v = refs[K + 1], refs[K + 2], refs[K + 3]
    sems = refs[K + 4]

    wid = lax.axis_index("s") * NC + lax.axis_index("c")
    bbase = wid * ROWS_PER_W

    pltpu.sync_copy(x_hbm.at[:, pl.ds(bbase, ROWS_PER_W)], idx_v)

    zero = jnp.zeros((LANES,), jnp.float32)

    def zrow(r, carry):
      for b in range(NACC):
        for c in range(D_VREGS):
          acc_v[b, r, pl.ds(c * LANES, LANES)] = zero
      return carry

    lax.fori_loop(0, ROWS_PER_W, zrow, 0)

    def mkidx(h, carry):
      for v in range(RV):
        raw = idx_v[h, pl.ds(v * LANES, LANES)]
        for ck in range(K):
          rel = raw - CBASE[ck]
          ok = plsc.bitcast(rel, jnp.uint32) < jnp.uint32(CHS[ck])
          idxk_v[ck, h, pl.ds(v * LANES, LANES)] = jnp.where(ok, rel, CHS[ck])
      return carry

    lax.fori_loop(0, H, mkidx, 0)

    j = 0
    for h in range(H):
      for ck in range(K):
        b = j % NACC
        if j >= NACC:
          pltpu.make_async_copy(
              tabs[ck].at[plsc.Indices(idxk_v.at[ck, h], ignored_value=CHS[ck])],
              acc_v.at[b], sems[b],
          ).wait()
        pltpu.async_copy(
            tabs[ck].at[plsc.Indices(idxk_v.at[ck, h], ignored_value=CHS[ck])],
            acc_v.at[b], sems[b], add=True,
        )
        j += 1
    for b in range(NACC):
      pltpu.make_async_copy(
          tabs[0].at[plsc.Indices(idxk_v.at[0, 0], ignored_value=CHS[0])],
          acc_v.at[b], sems[b],
      ).wait()

    def combine(r, carry):
      for c in range(D_VREGS):
        s = acc_v[0, r, pl.ds(c * LANES, LANES)]
        for b in range(1, NACC):
          s = s + acc_v[b, r, pl.ds(c * LANES, LANES)]
        acc_v[0, r, pl.ds(c * LANES, LANES)] = s
      return carry

    lax.fori_loop(0, ROWS_PER_W, combine, 0)
    pltpu.sync_copy(acc_v.at[0], out_hbm.at[pl.ds(bbase, ROWS_PER_W)])

  return k(xt, *chunks)


def _normalize(sums, t2):
  def body(s_ref, t_ref, o_ref):
    del t_ref
    p = s_ref[...] * (1.0 / H)
    ss = jnp.sum(p * p, axis=1, keepdims=True)
    denom = jnp.maximum(jnp.sqrt(ss), 1e-12)
    o_ref[...] = p / denom

  return pl.pallas_call(
      body,
      grid=(1,),
      in_specs=[
          pl.BlockSpec((B, D), lambda i: (0, 0)),
          pl.BlockSpec((8, D), lambda i: (0, 0)),
      ],
      out_specs=pl.BlockSpec((B, D), lambda i: (0, 0)),
      out_shape=jax.ShapeDtypeStruct((B, D), jnp.float32),
  )(sums, t2)


@jax.jit
def kernel(x, table):
  xt = x.astype(jnp.int32).T
  t2 = lax.optimization_barrier(table)
  chunks = [lax.slice(table, (CBASE[0], 0), (CBASE[0] + CHS[0], D))]
  chunks += [
      lax.slice(t2, (CBASE[ck], 0), (CBASE[ck] + CHS[ck], D))
      for ck in range(1, K)
  ]
  sums = _sc_pool_sums(xt, chunks)
  return _normalize(sums, t2)


# speedup vs baseline: 1.7066x; 1.7066x over previous
import functools

import jax
import jax.numpy as jnp
from jax import lax
from jax.experimental import pallas as pl
from jax.experimental.pallas import tpu as pltpu
from jax.experimental.pallas import tpu_sc as plsc

VOCAB = 1000000
D = 64
B = 4096
H = 50
LANES = 16
D_VREGS = D // LANES

NC = 2
NS = 16
NW = NC * NS
ROWS_PER_W = B // NW
RV = ROWS_PER_W // LANES
NACC = 8


def _sc_tower(xt, table):
  mesh = plsc.VectorSubcoreMesh(
      core_axis_name="c", subcore_axis_name="s", num_cores=NC, num_subcores=NS
  )

  @functools.partial(
      pl.kernel,
      out_type=jax.ShapeDtypeStruct((B, D), jnp.float32),
      mesh=mesh,
      compiler_params=pltpu.CompilerParams(
          use_tc_tiling_on_sc=False, needs_layout_passes=False
      ),
      scratch_types=[
          pltpu.VMEM((H, ROWS_PER_W), jnp.int32),
          pltpu.VMEM((NACC, ROWS_PER_W, D), jnp.float32),
          pltpu.VMEM((ROWS_PER_W, D), jnp.float32),
          pltpu.VMEM((ROWS_PER_W, LANES), jnp.float32),
          pltpu.VMEM((ROWS_PER_W,), jnp.float32),
          [pltpu.SemaphoreType.DMA] * NACC,
      ],
  )
  def k(x_hbm, tab_hbm, out_hbm, idx_v, acc_v, out_v, sq_v, nrm_v, sems):
    wid = lax.axis_index("s") * NC + lax.axis_index("c")
    bbase = wid * ROWS_PER_W

    pltpu.sync_copy(x_hbm.at[:, pl.ds(bbase, ROWS_PER_W)], idx_v)

    for h in range(H):
      b = h % NACC
      if h >= NACC:
        pltpu.make_async_copy(
            tab_hbm.at[idx_v.at[h]], acc_v.at[b], sems[b]
        ).wait()
      pltpu.async_copy(
          tab_hbm.at[idx_v.at[h]], acc_v.at[b], sems[b], add=(h >= NACC)
      )
    for b in range(NACC):
      pltpu.make_async_copy(tab_hbm.at[idx_v.at[b]], acc_v.at[b], sems[b]).wait()

    def combine(r, carry):
      sq = None
      for c in range(D_VREGS):
        s = acc_v[0, r, pl.ds(c * LANES, LANES)]
        for b in range(1, NACC):
          s = s + acc_v[b, r, pl.ds(c * LANES, LANES)]
        p = s * (1.0 / H)
        out_v[r, pl.ds(c * LANES, LANES)] = p
        sq = p * p if sq is None else sq + p * p
      sq_v[r] = sq
      return carry

    lax.fori_loop(0, ROWS_PER_W, combine, 0)

    rows16 = lax.iota(jnp.int32, LANES)

    for v in range(RV):
      n = None
      for c in range(LANES):
        g = plsc.load_gather(
            sq_v, [rows16 + (v * LANES), jnp.full((LANES,), c, jnp.int32)]
        )
        n = g if n is None else n + g
      i = plsc.bitcast(n, jnp.int32)
      y = plsc.bitcast(jnp.int32(0x5F3759DF) - (i >> 1), jnp.float32)
      for _ in range(3):
        y = y * (1.5 - 0.5 * n * y * y)
      nrm_v[pl.ds(v * LANES, LANES)] = jnp.minimum(y, 1e12)

    for v in range(RV):
      nv = nrm_v[pl.ds(v * LANES, LANES)]
      for i in range(LANES):
        r = v * LANES + i
        f = jnp.full((LANES,), nv[i], jnp.float32)
        for c in range(D_VREGS):
          out_v[r, pl.ds(c * LANES, LANES)] = (
              out_v[r, pl.ds(c * LANES, LANES)] * f
          )
    pltpu.sync_copy(out_v, out_hbm.at[pl.ds(bbase, ROWS_PER_W)])

  return k(xt, table)


@jax.jit
def kernel(x, table):
  xt = x.astype(jnp.int32).T
  return _sc_tower(xt, table)
